# baseline (speedup 1.0000x reference)
import functools

import jax
import jax.numpy as jnp
from jax import lax
from jax.experimental import pallas as pl
from jax.experimental.pallas import tpu as pltpu
from jax.experimental.pallas import tpu_sc as plsc

NC = 2
NS = 16
NW = NC * NS
LANES = 16
CH = 128


def _sc_mesh():
    return plsc.VectorSubcoreMesh(core_axis_name="c", subcore_axis_name="s")


def _sc_params():
    return pltpu.CompilerParams(needs_layout_passes=False)


@functools.lru_cache(maxsize=None)
def _deg_fn(ep, ndeg):
    njd = ep // NW // CH

    @functools.partial(
        pl.kernel,
        out_type=jax.ShapeDtypeStruct((NW, ndeg), jnp.float32),
        mesh=_sc_mesh(),
        compiler_params=_sc_params(),
        scratch_types=[
            pltpu.VMEM((njd, CH), jnp.int32),
            pltpu.VMEM((ndeg,), jnp.float32),
        ],
    )
    def deg_kernel(dst_hbm, out_hbm, dst_v, deg_v):
        c = lax.axis_index("c")
        s = lax.axis_index("s")
        w = s * NC + c
        pltpu.sync_copy(dst_hbm.at[w], dst_v)
        zero16 = jnp.zeros((LANES,), jnp.float32)
        ones16 = jnp.ones((LANES,), jnp.float32)

        def zb(i, carry):
            deg_v[pl.ds(i * LANES, LANES)] = zero16
            return carry

        lax.fori_loop(0, ndeg // LANES, zb, 0)

        def body(j, carry):
            for k in range(CH // LANES):
                idx = dst_v[j, pl.ds(k * LANES, LANES)]
                plsc.addupdate_scatter(deg_v, [idx], ones16)
            return carry

        lax.fori_loop(0, njd, body, 0)
        pltpu.sync_copy(deg_v, out_hbm.at[w])

    return deg_kernel


@functools.lru_cache(maxsize=None)
def _scatter_fn(ep, n, d, ea, eb):
    chs = 64
    hp = ((n + 1 + NS * 8 - 1) // (NS * 8)) * NS * 8
    rt = hp // NS
    emax = max(ea, eb)
    assert 16 * (ea + eb) == ep
    assert ea % (2 * chs) == 0 and eb % (2 * chs) == 0 and rt % 8 == 0

    @functools.partial(
        pl.kernel,
        out_type=jax.ShapeDtypeStruct((NC, hp, d), jnp.float32),
        mesh=_sc_mesh(),
        compiler_params=_sc_params(),
        scratch_types=[
            pltpu.VMEM((emax,), jnp.int32),
            pltpu.VMEM((emax,), jnp.int32),
            pltpu.VMEM((chs, d), jnp.float32),
            pltpu.VMEM((chs, d), jnp.float32),
            pltpu.VMEM_SHARED((hp, d), jnp.float32),
            pltpu.SemaphoreType.DMA,
            pltpu.SemaphoreType.DMA,
        ],
    )
    def scat_kernel(t_hbm, src_hbm, dst_hbm, zro_hbm, out_hbm,
                    src_v, dst_v, rows_a, rows_b, acc, sem_a, sem_b):
        c = lax.axis_index("c")
        s = lax.axis_index("s")

        def gather(j, buf, sem):
            pltpu.async_copy(
                t_hbm.at[src_v.at[pl.ds(j * chs, chs)]], buf, sem)

        def gwait(buf, sem):
            pltpu.make_async_copy(
                t_hbm.at[src_v.at[pl.ds(0, chs)]], buf, sem).wait()

        def scat(j, buf):
            pltpu.sync_copy(buf, acc.at[dst_v.at[pl.ds(j * chs, chs)]],
                            add=True)

        def run(esz, off):
            nj = esz // chs
            pltpu.sync_copy(src_hbm.at[pl.ds(off, esz)],
                            src_v.at[pl.ds(0, esz)])
            pltpu.sync_copy(dst_hbm.at[pl.ds(off, esz)],
                            dst_v.at[pl.ds(0, esz)])
            pltpu.sync_copy(zro_hbm.at[pl.ds(s * rt, rt)],
                            acc.at[pl.ds(s * rt, rt)])
            plsc.subcore_barrier()
            gather(0, rows_a, sem_a)

            def body(i, carry):
                j0 = 2 * i
                gather(j0 + 1, rows_b, sem_b)
                gwait(rows_a, sem_a)
                scat(j0, rows_a)

                @pl.when(j0 + 2 < nj)
                def _():
                    gather(j0 + 2, rows_a, sem_a)

                gwait(rows_b, sem_b)
                scat(j0 + 1, rows_b)
                return carry

            lax.fori_loop(0, nj // 2, body, 0)
            plsc.subcore_barrier()
            pltpu.sync_copy(acc.at[pl.ds(s * rt, rt)],
                            out_hbm.at[c, pl.ds(s * rt, rt)])

        @pl.when(c == 0)
        def _():
            run(ea, s * ea)

        @pl.when(c == 1)
        def _():
            run(eb, NS * ea + s * eb)

    return scat_kernel


def _tc_dinv(deg_partial, ndeg):
    bl = 128
    g = ndeg // bl

    def kfn(dp_ref, out_ref):
        sdeg = jnp.sum(dp_ref[...], axis=0, keepdims=True) + 1.0
        out_ref[...] = lax.rsqrt(sdeg)

    return pl.pallas_call(
        kfn,
        grid=(g,),
        in_specs=[pl.BlockSpec((NW, bl), lambda i: (0, i))],
        out_specs=pl.BlockSpec((1, bl), lambda i: (0, i)),
        out_shape=jax.ShapeDtypeStruct((1, ndeg), jnp.float32),
    )(deg_partial)


def _tc_first(x, w_mat, dinv2d):
    n, d = x.shape
    br = 1000

    def kfn(h_ref, w_ref, dv_ref, o_ref):
        t = jnp.dot(h_ref[...], w_ref[...], preferred_element_type=jnp.float32)
        o_ref[...] = t * dv_ref[...]

    return pl.pallas_call(
        kfn,
        grid=(n // br,),
        in_specs=[
            pl.BlockSpec((br, d), lambda i: (i, 0)),
            pl.BlockSpec((d, d), lambda i: (0, 0)),
            pl.BlockSpec((br, 1), lambda i: (i, 0)),
        ],
        out_specs=pl.BlockSpec((br, d), lambda i: (i, 0)),
        out_shape=jax.ShapeDtypeStruct((n, d), jnp.float32),
    )(x, w_mat, dinv2d)


def _tc_mid(acc0, acc1, tp, dinv2d, brow, w_mat):
    n, d = tp.shape
    br = 1000

    def kfn(a_ref, a2_ref, t_ref, dv_ref, bias_ref, w_ref, pre_ref, o_ref):
        pre = ((a_ref[...] + a2_ref[...] + t_ref[...]) * dv_ref[...]
               + bias_ref[...])
        pre_ref[...] = pre
        h = jnp.maximum(pre, 0.0)
        t = jnp.dot(h, w_ref[...], preferred_element_type=jnp.float32)
        o_ref[...] = t * dv_ref[...]

    return pl.pallas_call(
        kfn,
        grid=(n // br,),
        in_specs=[
            pl.BlockSpec((br, d), lambda i: (i, 0)),
            pl.BlockSpec((br, d), lambda i: (i, 0)),
            pl.BlockSpec((br, d), lambda i: (i, 0)),
            pl.BlockSpec((br, 1), lambda i: (i, 0)),
            pl.BlockSpec((1, d), lambda i: (0, 0)),
            pl.BlockSpec((d, d), lambda i: (0, 0)),
        ],
        out_specs=[
            pl.BlockSpec((br, d), lambda i: (i, 0)),
            pl.BlockSpec((br, d), lambda i: (i, 0)),
        ],
        out_shape=[
            jax.ShapeDtypeStruct((n, d), jnp.float32),
            jax.ShapeDtypeStruct((n, d), jnp.float32),
        ],
    )(acc0, acc1, tp, dinv2d, brow, w_mat)


def kernel(x, edge_index, W1, b1, W2, b2, W3, b3):
    n, d = x.shape
    e = edge_index.shape[1]
    ep = pl.cdiv(e, NW * CH) * NW * CH
    ndeg = ((n + CH) // CH) * CH

    pad = ep - e
    src_pad = jnp.concatenate(
        [edge_index[0], jnp.zeros((pad,), edge_index.dtype)])
    dst_pad = jnp.concatenate(
        [edge_index[1], jnp.full((pad,), n, edge_index.dtype)])
    njd = ep // NW // CH
    dstdeg = dst_pad.reshape(NW, njd, CH)

    degp = _deg_fn(ep, ndeg)(dstdeg)
    dinv2d = _tc_dinv(degp, ndeg)[0, :n].reshape(n, 1)
    b1r, b2r, b3r = (b.reshape(1, d) for b in (b1, b2, b3))

    epw = ep // NW
    ea = (int(2 * epw * 0.32) // 128) * 128
    eb = 2 * epw - ea
    hp = ((n + 1 + NS * 8 - 1) // (NS * 8)) * NS * 8
    zro = jnp.zeros((hp, d), jnp.float32)
    scat = _scatter_fn(ep, n, d, ea, eb)

    def run_scatter(t):
        parts = scat(t, src_pad, dst_pad, zro)
        return parts[0, :n], parts[1, :n]

    t1 = _tc_first(x, W1, dinv2d)

    w_stack = jnp.stack([W2, W3, jnp.eye(d, dtype=x.dtype)])
    b_stack = jnp.stack([b1r, b2r, b3r])

    def step(t, xs):
        w_l, b_l = xs
        a0, a1 = run_scatter(t)
        pre, t_next = _tc_mid(a0, a1, t, dinv2d, b_l, w_l)
        return t_next, pre

    _, pres = lax.scan(step, t1, (w_stack, b_stack))
    return pres[-1]


# speedup vs baseline: 1.2279x; 1.2279x over previous
import functools

import jax
import jax.numpy as jnp
from jax import lax
from jax.experimental import pallas as pl
from jax.experimental.pallas import tpu as pltpu
from jax.experimental.pallas import tpu_sc as plsc

NC = 2
NS = 16
NW = NC * NS
LANES = 16
CH = 128


def _sc_mesh():
    return plsc.VectorSubcoreMesh(core_axis_name="c", subcore_axis_name="s")


def _sc_params():
    return pltpu.CompilerParams(needs_layout_passes=False)


@functools.lru_cache(maxsize=None)
def _deg_fn(ep, ndeg):
    njd = ep // NW // CH

    @functools.partial(
        pl.kernel,
        out_type=jax.ShapeDtypeStruct((NW, ndeg), jnp.float32),
        mesh=_sc_mesh(),
        compiler_params=_sc_params(),
        scratch_types=[
            pltpu.VMEM((njd, CH), jnp.int32),
            pltpu.VMEM((ndeg,), jnp.float32),
        ],
    )
    def deg_kernel(dst_hbm, out_hbm, dst_v, deg_v):
        c = lax.axis_index("c")
        s = lax.axis_index("s")
        w = s * NC + c
        pltpu.sync_copy(dst_hbm.at[w], dst_v)
        zero16 = jnp.zeros((LANES,), jnp.float32)
        ones16 = jnp.ones((LANES,), jnp.float32)

        def zb(i, carry):
            deg_v[pl.ds(i * LANES, LANES)] = zero16
            return carry

        lax.fori_loop(0, ndeg // LANES, zb, 0)

        def body(j, carry):
            for k in range(CH // LANES):
                idx = dst_v[j, pl.ds(k * LANES, LANES)]
                plsc.addupdate_scatter(deg_v, [idx], ones16)
            return carry

        lax.fori_loop(0, njd, body, 0)
        pltpu.sync_copy(deg_v, out_hbm.at[w])

    return deg_kernel


@functools.lru_cache(maxsize=None)
def _scatter_fn(ep, n, d, ea, eb):
    chs = 64
    hp = ((n + 1 + NS * 8 - 1) // (NS * 8)) * NS * 8
    rt = hp // NS
    emax = max(ea, eb)
    assert 16 * (ea + eb) == ep
    assert ea % (2 * chs) == 0 and eb % (2 * chs) == 0 and rt % 8 == 0

    @functools.partial(
        pl.kernel,
        out_type=jax.ShapeDtypeStruct((NC, hp, d), jnp.float32),
        mesh=_sc_mesh(),
        compiler_params=_sc_params(),
        scratch_types=[
            pltpu.VMEM((emax,), jnp.int32),
            pltpu.VMEM((emax,), jnp.int32),
            pltpu.VMEM((chs, d), jnp.float32),
            pltpu.VMEM((chs, d), jnp.float32),
            pltpu.VMEM_SHARED((hp, d), jnp.float32),
            pltpu.SemaphoreType.DMA,
            pltpu.SemaphoreType.DMA,
        ],
    )
    def scat_kernel(t_hbm, src_hbm, dst_hbm, zro_hbm, out_hbm,
                    src_v, dst_v, rows_a, rows_b, acc, sem_a, sem_b):
        c = lax.axis_index("c")
        s = lax.axis_index("s")

        def gather(j, buf, sem):
            pltpu.async_copy(
                t_hbm.at[src_v.at[pl.ds(j * chs, chs)]], buf, sem)

        def gwait(buf, sem):
            pltpu.make_async_copy(
                t_hbm.at[src_v.at[pl.ds(0, chs)]], buf, sem).wait()

        def scat(j, buf):
            pltpu.sync_copy(buf, acc.at[dst_v.at[pl.ds(j * chs, chs)]],
                            add=True)

        def run(esz, off):
            nj = esz // chs
            pltpu.sync_copy(src_hbm.at[pl.ds(off, esz)],
                            src_v.at[pl.ds(0, esz)])
            pltpu.sync_copy(dst_hbm.at[pl.ds(off, esz)],
                            dst_v.at[pl.ds(0, esz)])
            pltpu.sync_copy(zro_hbm.at[pl.ds(s * rt, rt)],
                            acc.at[pl.ds(s * rt, rt)])
            plsc.subcore_barrier()
            gather(0, rows_a, sem_a)

            def body(i, carry):
                j0 = 2 * i
                gather(j0 + 1, rows_b, sem_b)
                gwait(rows_a, sem_a)
                scat(j0, rows_a)

                @pl.when(j0 + 2 < nj)
                def _():
                    gather(j0 + 2, rows_a, sem_a)

                gwait(rows_b, sem_b)
                scat(j0 + 1, rows_b)
                return carry

            lax.fori_loop(0, nj // 2, body, 0)
            plsc.subcore_barrier()
            pltpu.sync_copy(acc.at[pl.ds(s * rt, rt)],
                            out_hbm.at[c, pl.ds(s * rt, rt)])

        @pl.when(c == 0)
        def _():
            run(ea, s * ea)

        @pl.when(c == 1)
        def _():
            run(eb, NS * ea + s * eb)

    return scat_kernel


def _tc_dinv(deg_partial, ndeg):
    bl = 128
    g = ndeg // bl

    def kfn(dp_ref, out_ref):
        sdeg = jnp.sum(dp_ref[...], axis=0, keepdims=True) + 1.0
        out_ref[...] = lax.rsqrt(sdeg)

    return pl.pallas_call(
        kfn,
        grid=(g,),
        in_specs=[pl.BlockSpec((NW, bl), lambda i: (0, i))],
        out_specs=pl.BlockSpec((1, bl), lambda i: (0, i)),
        out_shape=jax.ShapeDtypeStruct((1, ndeg), jnp.float32),
    )(deg_partial)


def _tc_first(x, w_mat, dinv2d):
    n, d = x.shape
    br = 1000

    def kfn(h_ref, w_ref, dv_ref, o_ref):
        t = jnp.dot(h_ref[...], w_ref[...], preferred_element_type=jnp.float32)
        o_ref[...] = t * dv_ref[...]

    return pl.pallas_call(
        kfn,
        grid=(n // br,),
        in_specs=[
            pl.BlockSpec((br, d), lambda i: (i, 0)),
            pl.BlockSpec((d, d), lambda i: (0, 0)),
            pl.BlockSpec((br, 1), lambda i: (i, 0)),
        ],
        out_specs=pl.BlockSpec((br, d), lambda i: (i, 0)),
        out_shape=jax.ShapeDtypeStruct((n, d), jnp.float32),
    )(x, w_mat, dinv2d)


def _tc_mid(acc0, acc1, tp, dinv2d, brow, w_mat):
    n, d = tp.shape
    br = 1000

    def kfn(a_ref, a2_ref, t_ref, dv_ref, bias_ref, w_ref, pre_ref, o_ref):
        pre = ((a_ref[...] + a2_ref[...] + t_ref[...]) * dv_ref[...]
               + bias_ref[...])
        pre_ref[...] = pre
        h = jnp.maximum(pre, 0.0)
        t = jnp.dot(h, w_ref[...], preferred_element_type=jnp.float32)
        o_ref[...] = t * dv_ref[...]

    return pl.pallas_call(
        kfn,
        grid=(n // br,),
        in_specs=[
            pl.BlockSpec((br, d), lambda i: (i, 0)),
            pl.BlockSpec((br, d), lambda i: (i, 0)),
            pl.BlockSpec((br, d), lambda i: (i, 0)),
            pl.BlockSpec((br, 1), lambda i: (i, 0)),
            pl.BlockSpec((1, d), lambda i: (0, 0)),
            pl.BlockSpec((d, d), lambda i: (0, 0)),
        ],
        out_specs=[
            pl.BlockSpec((br, d), lambda i: (i, 0)),
            pl.BlockSpec((br, d), lambda i: (i, 0)),
        ],
        out_shape=[
            jax.ShapeDtypeStruct((n, d), jnp.float32),
            jax.ShapeDtypeStruct((n, d), jnp.float32),
        ],
    )(acc0, acc1, tp, dinv2d, brow, w_mat)


def kernel(x, edge_index, W1, b1, W2, b2, W3, b3):
    n, d = x.shape
    e = edge_index.shape[1]
    ep = pl.cdiv(e, NW * CH) * NW * CH
    ndeg = ((n + CH) // CH) * CH

    pad = ep - e
    src_pad = jnp.concatenate(
        [edge_index[0], jnp.zeros((pad,), edge_index.dtype)])
    dst_pad = jnp.concatenate(
        [edge_index[1], jnp.full((pad,), n, edge_index.dtype)])
    njd = ep // NW // CH
    dstdeg = dst_pad.reshape(NW, njd, CH)

    degp = _deg_fn(ep, ndeg)(dstdeg)
    dinv2d = _tc_dinv(degp, ndeg)[0, :n].reshape(n, 1)
    b1r, b2r, b3r = (b.reshape(1, d) for b in (b1, b2, b3))

    epw = ep // NW
    ea = (int(2 * epw * 0.68) // 128) * 128
    eb = 2 * epw - ea
    hp = ((n + 1 + NS * 8 - 1) // (NS * 8)) * NS * 8
    zro = jnp.zeros((hp, d), jnp.float32)
    scat = _scatter_fn(ep, n, d, ea, eb)

    def run_scatter(t):
        parts = scat(t, src_pad, dst_pad, zro)
        return parts[0, :n], parts[1, :n]

    t1 = _tc_first(x, W1, dinv2d)

    w_stack = jnp.stack([W2, W3, jnp.eye(d, dtype=x.dtype)])
    b_stack = jnp.stack([b1r, b2r, b3r])

    def step(t, xs):
        w_l, b_l = xs
        a0, a1 = run_scatter(t)
        pre, t_next = _tc_mid(a0, a1, t, dinv2d, b_l, w_l)
        return t_next, pre

    _, pres = lax.scan(step, t1, (w_stack, b_stack))
    return pres[-1]


# speedup vs baseline: 1.2938x; 1.0537x over previous
import functools

import jax
import jax.numpy as jnp
from jax import lax
from jax.experimental import pallas as pl
from jax.experimental.pallas import tpu as pltpu
from jax.experimental.pallas import tpu_sc as plsc

NC = 2
NS = 16
NW = NC * NS
LANES = 16
CH = 128


def _sc_mesh():
    return plsc.VectorSubcoreMesh(core_axis_name="c", subcore_axis_name="s")


def _sc_params():
    return pltpu.CompilerParams(needs_layout_passes=False)


@functools.lru_cache(maxsize=None)
def _deg_fn(ep, ndeg):
    njd = ep // NW // CH

    @functools.partial(
        pl.kernel,
        out_type=jax.ShapeDtypeStruct((NW, ndeg), jnp.float32),
        mesh=_sc_mesh(),
        compiler_params=_sc_params(),
        scratch_types=[
            pltpu.VMEM((njd, CH), jnp.int32),
            pltpu.VMEM((ndeg,), jnp.float32),
        ],
    )
    def deg_kernel(dst_hbm, out_hbm, dst_v, deg_v):
        c = lax.axis_index("c")
        s = lax.axis_index("s")
        w = s * NC + c
        pltpu.sync_copy(dst_hbm.at[w], dst_v)
        zero16 = jnp.zeros((LANES,), jnp.float32)
        ones16 = jnp.ones((LANES,), jnp.float32)

        def zb(i, carry):
            deg_v[pl.ds(i * LANES, LANES)] = zero16
            return carry

        lax.fori_loop(0, ndeg // LANES, zb, 0)

        def body(j, carry):
            for k in range(CH // LANES):
                idx = dst_v[j, pl.ds(k * LANES, LANES)]
                plsc.addupdate_scatter(deg_v, [idx], ones16)
            return carry

        lax.fori_loop(0, njd, body, 0)
        pltpu.sync_copy(deg_v, out_hbm.at[w])

    return deg_kernel


@functools.lru_cache(maxsize=None)
def _scatter_fn(ep, n, d, ea, eb):
    chs = 64
    hp = ((n + 1 + NS * 8 - 1) // (NS * 8)) * NS * 8
    rt = hp // NS
    emax = max(ea, eb)
    assert 16 * (ea + eb) == ep
    assert ea % (2 * chs) == 0 and eb % (2 * chs) == 0 and rt % 8 == 0

    @functools.partial(
        pl.kernel,
        out_type=jax.ShapeDtypeStruct((NC, hp, d), jnp.float32),
        mesh=_sc_mesh(),
        compiler_params=_sc_params(),
        scratch_types=[
            pltpu.VMEM((emax,), jnp.int32),
            pltpu.VMEM((emax,), jnp.int32),
            pltpu.VMEM((chs, d), jnp.float32),
            pltpu.VMEM((chs, d), jnp.float32),
            pltpu.VMEM_SHARED((hp, d), jnp.float32),
            pltpu.SemaphoreType.DMA,
            pltpu.SemaphoreType.DMA,
        ],
    )
    def scat_kernel(t_hbm, src_hbm, dst_hbm, zro_hbm, out_hbm,
                    src_v, dst_v, rows_a, rows_b, acc, sem_a, sem_b):
        c = lax.axis_index("c")
        s = lax.axis_index("s")

        def gather(j, buf, sem):
            pltpu.async_copy(
                t_hbm.at[src_v.at[pl.ds(j * chs, chs)]], buf, sem)

        def gwait(buf, sem):
            pltpu.make_async_copy(
                t_hbm.at[src_v.at[pl.ds(0, chs)]], buf, sem).wait()

        def scat(j, buf):
            pltpu.sync_copy(buf, acc.at[dst_v.at[pl.ds(j * chs, chs)]],
                            add=True)

        def run(esz, off):
            nj = esz // chs
            pltpu.sync_copy(src_hbm.at[pl.ds(off, esz)],
                            src_v.at[pl.ds(0, esz)])
            pltpu.sync_copy(dst_hbm.at[pl.ds(off, esz)],
                            dst_v.at[pl.ds(0, esz)])
            pltpu.sync_copy(zro_hbm.at[pl.ds(s * rt, rt)],
                            acc.at[pl.ds(s * rt, rt)])
            plsc.subcore_barrier()
            gather(0, rows_a, sem_a)

            def body(i, carry):
                j0 = 2 * i
                gather(j0 + 1, rows_b, sem_b)
                gwait(rows_a, sem_a)
                scat(j0, rows_a)

                @pl.when(j0 + 2 < nj)
                def _():
                    gather(j0 + 2, rows_a, sem_a)

                gwait(rows_b, sem_b)
                scat(j0 + 1, rows_b)
                return carry

            lax.fori_loop(0, nj // 2, body, 0)
            plsc.subcore_barrier()
            pltpu.sync_copy(acc.at[pl.ds(s * rt, rt)],
                            out_hbm.at[c, pl.ds(s * rt, rt)])

        @pl.when(c == 0)
        def _():
            run(ea, s * ea)

        @pl.when(c == 1)
        def _():
            run(eb, NS * ea + s * eb)

    return scat_kernel


def _tc_dinv(deg_partial, ndeg):
    def kfn(dp_ref, out_ref):
        sdeg = jnp.sum(dp_ref[...], axis=0, keepdims=True) + 1.0
        out_ref[...] = lax.rsqrt(sdeg)

    return pl.pallas_call(
        kfn,
        out_shape=jax.ShapeDtypeStruct((1, ndeg), jnp.float32),
    )(deg_partial)


def _tc_first(x, w_mat, dinv2d):
    n, d = x.shape
    br = 1000

    def kfn(h_ref, w_ref, dv_ref, o_ref):
        t = jnp.dot(h_ref[...], w_ref[...], preferred_element_type=jnp.float32)
        o_ref[...] = t * dv_ref[...]

    return pl.pallas_call(
        kfn,
        grid=(n // br,),
        in_specs=[
            pl.BlockSpec((br, d), lambda i: (i, 0)),
            pl.BlockSpec((d, d), lambda i: (0, 0)),
            pl.BlockSpec((br, 1), lambda i: (i, 0)),
        ],
        out_specs=pl.BlockSpec((br, d), lambda i: (i, 0)),
        out_shape=jax.ShapeDtypeStruct((n, d), jnp.float32),
    )(x, w_mat, dinv2d)


def _tc_mid(parts, tp, dinv2d, brow, w_mat):
    n, d = tp.shape
    br = 1000

    def kfn(a_ref, a2_ref, t_ref, dv_ref, bias_ref, w_ref, pre_ref, o_ref):
        pre = ((a_ref[0] + a2_ref[0] + t_ref[...]) * dv_ref[...]
               + bias_ref[...])
        pre_ref[...] = pre
        h = jnp.maximum(pre, 0.0)
        t = jnp.dot(h, w_ref[...], preferred_element_type=jnp.float32)
        o_ref[...] = t * dv_ref[...]

    return pl.pallas_call(
        kfn,
        grid=(n // br,),
        in_specs=[
            pl.BlockSpec((1, br, d), lambda i: (0, i, 0)),
            pl.BlockSpec((1, br, d), lambda i: (1, i, 0)),
            pl.BlockSpec((br, d), lambda i: (i, 0)),
            pl.BlockSpec((br, 1), lambda i: (i, 0)),
            pl.BlockSpec((1, d), lambda i: (0, 0)),
            pl.BlockSpec((d, d), lambda i: (0, 0)),
        ],
        out_specs=[
            pl.BlockSpec((br, d), lambda i: (i, 0)),
            pl.BlockSpec((br, d), lambda i: (i, 0)),
        ],
        out_shape=[
            jax.ShapeDtypeStruct((n, d), jnp.float32),
            jax.ShapeDtypeStruct((n, d), jnp.float32),
        ],
    )(parts, parts, tp, dinv2d, brow, w_mat)


def kernel(x, edge_index, W1, b1, W2, b2, W3, b3):
    n, d = x.shape
    e = edge_index.shape[1]
    ep = pl.cdiv(e, NW * CH) * NW * CH
    ndeg = ((n + CH) // CH) * CH

    pad = ep - e
    src_pad = jnp.concatenate(
        [edge_index[0], jnp.zeros((pad,), edge_index.dtype)])
    dst_pad = jnp.concatenate(
        [edge_index[1], jnp.full((pad,), n, edge_index.dtype)])
    njd = ep // NW // CH
    dstdeg = dst_pad.reshape(NW, njd, CH)

    degp = _deg_fn(ep, ndeg)(dstdeg)
    dinv2d = _tc_dinv(degp, ndeg)[0, :n].reshape(n, 1)
    b1r, b2r, b3r = (b.reshape(1, d) for b in (b1, b2, b3))

    epw = ep // NW
    ea = (int(2 * epw * 0.62) // 128) * 128
    eb = 2 * epw - ea
    hp = ((n + 1 + NS * 8 - 1) // (NS * 8)) * NS * 8
    zro = jnp.zeros((hp, d), jnp.float32)
    scat = _scatter_fn(ep, n, d, ea, eb)

    def run_scatter(t):
        return scat(t, src_pad, dst_pad, zro)

    t1 = _tc_first(x, W1, dinv2d)

    w_stack = jnp.stack([W2, W3, jnp.eye(d, dtype=x.dtype)])
    b_stack = jnp.stack([b1r, b2r, b3r])

    def step(t, xs):
        w_l, b_l = xs
        parts = run_scatter(t)
        pre, t_next = _tc_mid(parts, t, dinv2d, b_l, w_l)
        return t_next, pre

    _, pres = lax.scan(step, t1, (w_stack, b_stack))
    return pres[-1]


# speedup vs baseline: 1.3331x; 1.0304x over previous
import functools

import jax
import jax.numpy as jnp
from jax import lax
from jax.experimental import pallas as pl
from jax.experimental.pallas import tpu as pltpu
from jax.experimental.pallas import tpu_sc as plsc

NC = 2
NS = 16
NW = NC * NS
LANES = 16
CH = 128


def _sc_mesh():
    return plsc.VectorSubcoreMesh(core_axis_name="c", subcore_axis_name="s")


def _sc_params():
    return pltpu.CompilerParams(needs_layout_passes=False)


@functools.lru_cache(maxsize=None)
def _deg_fn(ep, ndeg):
    njd = ep // NW // CH

    @functools.partial(
        pl.kernel,
        out_type=jax.ShapeDtypeStruct((NW, ndeg), jnp.float32),
        mesh=_sc_mesh(),
        compiler_params=_sc_params(),
        scratch_types=[
            pltpu.VMEM((njd, CH), jnp.int32),
            pltpu.VMEM((ndeg,), jnp.float32),
        ],
    )
    def deg_kernel(dst_hbm, out_hbm, dst_v, deg_v):
        c = lax.axis_index("c")
        s = lax.axis_index("s")
        w = s * NC + c
        pltpu.sync_copy(dst_hbm.at[w], dst_v)
        zero16 = jnp.zeros((LANES,), jnp.float32)
        ones16 = jnp.ones((LANES,), jnp.float32)

        def zb(i, carry):
            deg_v[pl.ds(i * LANES, LANES)] = zero16
            return carry

        lax.fori_loop(0, ndeg // LANES, zb, 0)

        def body(j, carry):
            for k in range(CH // LANES):
                idx = dst_v[j, pl.ds(k * LANES, LANES)]
                plsc.addupdate_scatter(deg_v, [idx], ones16)
            return carry

        lax.fori_loop(0, njd, body, 0)
        pltpu.sync_copy(deg_v, out_hbm.at[w])

    return deg_kernel


@functools.lru_cache(maxsize=None)
def _scatter_fn(ep, n, d, ea, eb):
    chs = 64
    hp = ((n + 1 + NS * 8 - 1) // (NS * 8)) * NS * 8
    rt = hp // NS
    emax = max(ea, eb)
    assert 16 * (ea + eb) == ep
    assert ea % (2 * chs) == 0 and eb % (2 * chs) == 0 and rt % 8 == 0

    @functools.partial(
        pl.kernel,
        out_type=jax.ShapeDtypeStruct((NC, hp, d), jnp.float32),
        mesh=_sc_mesh(),
        compiler_params=_sc_params(),
        scratch_types=[
            pltpu.VMEM((emax,), jnp.int32),
            pltpu.VMEM((emax,), jnp.int32),
            pltpu.VMEM((chs, d), jnp.float32),
            pltpu.VMEM((chs, d), jnp.float32),
            pltpu.VMEM_SHARED((hp, d), jnp.float32),
            pltpu.SemaphoreType.DMA,
            pltpu.SemaphoreType.DMA,
        ],
    )
    def scat_kernel(t_hbm, src_hbm, dst_hbm, zro_hbm, out_hbm,
                    src_v, dst_v, rows_a, rows_b, acc, sem_a, sem_b):
        c = lax.axis_index("c")
        s = lax.axis_index("s")

        def gather(j, buf, sem):
            pltpu.async_copy(
                t_hbm.at[src_v.at[pl.ds(j * chs, chs)]], buf, sem)

        def gwait(buf, sem):
            pltpu.make_async_copy(
                t_hbm.at[src_v.at[pl.ds(0, chs)]], buf, sem).wait()

        def scat(j, buf):
            pltpu.sync_copy(buf, acc.at[dst_v.at[pl.ds(j * chs, chs)]],
                            add=True)

        def run(esz, off):
            nj = esz // chs
            pltpu.sync_copy(src_hbm.at[pl.ds(off, esz)],
                            src_v.at[pl.ds(0, esz)])
            pltpu.sync_copy(dst_hbm.at[pl.ds(off, esz)],
                            dst_v.at[pl.ds(0, esz)])
            pltpu.sync_copy(zro_hbm.at[pl.ds(s * rt, rt)],
                            acc.at[pl.ds(s * rt, rt)])
            plsc.subcore_barrier()
            gather(0, rows_a, sem_a)

            def body(i, carry):
                j0 = 2 * i
                gather(j0 + 1, rows_b, sem_b)
                gwait(rows_a, sem_a)
                scat(j0, rows_a)

                @pl.when(j0 + 2 < nj)
                def _():
                    gather(j0 + 2, rows_a, sem_a)

                gwait(rows_b, sem_b)
                scat(j0 + 1, rows_b)
                return carry

            lax.fori_loop(0, nj // 2, body, 0)
            plsc.subcore_barrier()
            pltpu.sync_copy(acc.at[pl.ds(s * rt, rt)],
                            out_hbm.at[c, pl.ds(s * rt, rt)])

        @pl.when(c == 0)
        def _():
            run(ea, s * ea)

        @pl.when(c == 1)
        def _():
            run(eb, NS * ea + s * eb)

    return scat_kernel


def _tc_dinv(deg_partial, ndeg):
    def kfn(dp_ref, out_ref):
        sdeg = jnp.sum(dp_ref[...], axis=0, keepdims=True) + 1.0
        out_ref[...] = lax.rsqrt(sdeg)

    return pl.pallas_call(
        kfn,
        out_shape=jax.ShapeDtypeStruct((1, ndeg), jnp.float32),
    )(deg_partial)


def _tc_first(x, w_mat, dinv2d):
    n, d = x.shape
    br = 1000

    def kfn(h_ref, w_ref, dv_ref, o_ref):
        t = jnp.dot(h_ref[...], w_ref[...], preferred_element_type=jnp.float32)
        o_ref[...] = t * dv_ref[...]

    return pl.pallas_call(
        kfn,
        grid=(n // br,),
        in_specs=[
            pl.BlockSpec((br, d), lambda i: (i, 0)),
            pl.BlockSpec((d, d), lambda i: (0, 0)),
            pl.BlockSpec((br, 1), lambda i: (i, 0)),
        ],
        out_specs=pl.BlockSpec((br, d), lambda i: (i, 0)),
        out_shape=jax.ShapeDtypeStruct((n, d), jnp.float32),
    )(x, w_mat, dinv2d)


def _tc_mid(parts, tp, dinv2d, brow, w_mat):
    n, d = tp.shape
    br = 1000

    def kfn(a_ref, a2_ref, t_ref, dv_ref, bias_ref, w_ref, pre_ref, o_ref):
        pre = ((a_ref[0] + a2_ref[0] + t_ref[...]) * dv_ref[...]
               + bias_ref[...])
        pre_ref[...] = pre
        h = jnp.maximum(pre, 0.0)
        t = jnp.dot(h, w_ref[...], preferred_element_type=jnp.float32)
        o_ref[...] = t * dv_ref[...]

    return pl.pallas_call(
        kfn,
        grid=(n // br,),
        in_specs=[
            pl.BlockSpec((1, br, d), lambda i: (0, i, 0)),
            pl.BlockSpec((1, br, d), lambda i: (1, i, 0)),
            pl.BlockSpec((br, d), lambda i: (i, 0)),
            pl.BlockSpec((br, 1), lambda i: (i, 0)),
            pl.BlockSpec((1, d), lambda i: (0, 0)),
            pl.BlockSpec((d, d), lambda i: (0, 0)),
        ],
        out_specs=[
            pl.BlockSpec((br, d), lambda i: (i, 0)),
            pl.BlockSpec((br, d), lambda i: (i, 0)),
        ],
        out_shape=[
            jax.ShapeDtypeStruct((n, d), jnp.float32),
            jax.ShapeDtypeStruct((n, d), jnp.float32),
        ],
    )(parts, parts, tp, dinv2d, brow, w_mat)


def kernel(x, edge_index, W1, b1, W2, b2, W3, b3):
    n, d = x.shape
    e = edge_index.shape[1]
    ep = pl.cdiv(e, NW * CH) * NW * CH
    ndeg = ((n + CH) // CH) * CH

    pad = ep - e
    hp0 = ((n + 1 + NS * 8 - 1) // (NS * 8)) * NS * 8
    pad_dst = n + jnp.arange(pad, dtype=edge_index.dtype) % (hp0 - n)
    src_pad = jnp.concatenate(
        [edge_index[0], jnp.zeros((pad,), edge_index.dtype)])
    dst_pad = jnp.concatenate([edge_index[1], pad_dst])
    njd = ep // NW // CH
    dstdeg = dst_pad.reshape(NW, njd, CH)

    degp = _deg_fn(ep, ndeg)(dstdeg)
    dinv2d = _tc_dinv(degp, ndeg)[0, :n].reshape(n, 1)
    b1r, b2r, b3r = (b.reshape(1, d) for b in (b1, b2, b3))

    epw = ep // NW
    ea = (int(2 * epw * 0.67) // 128) * 128
    eb = 2 * epw - ea
    hp = ((n + 1 + NS * 8 - 1) // (NS * 8)) * NS * 8
    zro = jnp.zeros((hp, d), jnp.float32)
    scat = _scatter_fn(ep, n, d, ea, eb)

    def run_scatter(t):
        return scat(t, src_pad, dst_pad, zro)

    t1 = _tc_first(x, W1, dinv2d)

    w_stack = jnp.stack([W2, W3, jnp.eye(d, dtype=x.dtype)])
    b_stack = jnp.stack([b1r, b2r, b3r])

    def step(t, xs):
        w_l, b_l = xs
        parts = run_scatter(t)
        pre, t_next = _tc_mid(parts, t, dinv2d, b_l, w_l)
        return t_next, pre

    _, pres = lax.scan(step, t1, (w_stack, b_stack))
    return pres[-1]


# speedup vs baseline: 1.4238x; 1.0680x over previous
import functools

import jax
import jax.numpy as jnp
from jax import lax
from jax.experimental import pallas as pl
from jax.experimental.pallas import tpu as pltpu
from jax.experimental.pallas import tpu_sc as plsc

NC = 2
NS = 16
NW = NC * NS
LANES = 16
CH = 128


def _sc_mesh():
    return plsc.VectorSubcoreMesh(core_axis_name="c", subcore_axis_name="s")


def _sc_params():
    return pltpu.CompilerParams(needs_layout_passes=False)


@functools.lru_cache(maxsize=None)
def _deg_fn(ep, ndeg):
    njd = ep // NW // CH

    @functools.partial(
        pl.kernel,
        out_type=jax.ShapeDtypeStruct((NW, ndeg), jnp.float32),
        mesh=_sc_mesh(),
        compiler_params=_sc_params(),
        scratch_types=[
            pltpu.VMEM((njd, CH), jnp.int32),
            pltpu.VMEM((ndeg,), jnp.float32),
        ],
    )
    def deg_kernel(dst_hbm, out_hbm, dst_v, deg_v):
        c = lax.axis_index("c")
        s = lax.axis_index("s")
        w = s * NC + c
        pltpu.sync_copy(dst_hbm.at[w], dst_v)
        zero16 = jnp.zeros((LANES,), jnp.float32)
        ones16 = jnp.ones((LANES,), jnp.float32)

        def zb(i, carry):
            deg_v[pl.ds(i * LANES, LANES)] = zero16
            return carry

        lax.fori_loop(0, ndeg // LANES, zb, 0)

        def body(j, carry):
            for k in range(CH // LANES):
                idx = dst_v[j, pl.ds(k * LANES, LANES)]
                plsc.addupdate_scatter(deg_v, [idx], ones16)
            return carry

        lax.fori_loop(0, njd, body, 0)
        pltpu.sync_copy(deg_v, out_hbm.at[w])

    return deg_kernel


@functools.lru_cache(maxsize=None)
def _scatter_fn(ep, n, d, ea, eb):
    chs = 64
    hp = ((n + 1 + NS * 8 - 1) // (NS * 8)) * NS * 8
    rt = hp // NS
    emax = max(ea, eb)
    assert 16 * (ea + eb) == ep
    assert ea % (2 * chs) == 0 and eb % (2 * chs) == 0 and rt % 8 == 0

    @functools.partial(
        pl.kernel,
        out_type=jax.ShapeDtypeStruct((NC, hp, d), jnp.float32),
        mesh=_sc_mesh(),
        compiler_params=_sc_params(),
        scratch_types=[
            pltpu.VMEM((emax,), jnp.int32),
            pltpu.VMEM((emax,), jnp.int32),
            pltpu.VMEM((chs, d), jnp.float32),
            pltpu.VMEM((chs, d), jnp.float32),
            pltpu.VMEM_SHARED((hp, d), jnp.float32),
            pltpu.SemaphoreType.DMA,
            pltpu.SemaphoreType.DMA,
        ],
    )
    def scat_kernel(t_hbm, src_hbm, dst_hbm, zro_hbm, out_hbm,
                    src_v, dst_v, rows_a, rows_b, acc, sem_a, sem_b):
        c = lax.axis_index("c")
        s = lax.axis_index("s")

        def gather(j, buf, sem):
            pltpu.async_copy(
                t_hbm.at[src_v.at[pl.ds(j * chs, chs)]], buf, sem)

        def gwait(buf, sem):
            pltpu.make_async_copy(
                t_hbm.at[src_v.at[pl.ds(0, chs)]], buf, sem).wait()

        def scat(j, buf):
            pltpu.sync_copy(buf, acc.at[dst_v.at[pl.ds(j * chs, chs)]],
                            add=True)

        def run(esz, off):
            nj = esz // chs
            pltpu.sync_copy(src_hbm.at[pl.ds(off, esz)],
                            src_v.at[pl.ds(0, esz)])
            pltpu.sync_copy(dst_hbm.at[pl.ds(off, esz)],
                            dst_v.at[pl.ds(0, esz)])
            pltpu.sync_copy(zro_hbm.at[pl.ds(s * rt, rt)],
                            acc.at[pl.ds(s * rt, rt)])
            plsc.subcore_barrier()
            gather(0, rows_a, sem_a)

            def body(i, carry):
                j0 = 2 * i
                gather(j0 + 1, rows_b, sem_b)
                gwait(rows_a, sem_a)
                scat(j0, rows_a)

                @pl.when(j0 + 2 < nj)
                def _():
                    gather(j0 + 2, rows_a, sem_a)

                gwait(rows_b, sem_b)
                scat(j0 + 1, rows_b)
                return carry

            lax.fori_loop(0, nj // 2, body, 0)
            plsc.subcore_barrier()
            pltpu.sync_copy(acc.at[pl.ds(s * rt, rt)],
                            out_hbm.at[c, pl.ds(s * rt, rt)])

        @pl.when(c == 0)
        def _():
            run(ea, s * ea)

        @pl.when(c == 1)
        def _():
            run(eb, NS * ea + s * eb)

    return scat_kernel


def _tc_dinv(deg_partial, ndeg):
    def kfn(dp_ref, out_ref):
        sdeg = jnp.sum(dp_ref[...], axis=0, keepdims=True) + 1.0
        out_ref[...] = lax.rsqrt(sdeg)

    return pl.pallas_call(
        kfn,
        out_shape=jax.ShapeDtypeStruct((1, ndeg), jnp.float32),
    )(deg_partial)


def _tc_first(x, w_mat, dinv2d):
    n, d = x.shape
    br = 1000

    def kfn(h_ref, w_ref, dv_ref, o_ref):
        t = jnp.dot(h_ref[...], w_ref[...], preferred_element_type=jnp.float32)
        o_ref[...] = t * dv_ref[...]

    return pl.pallas_call(
        kfn,
        grid=(n // br,),
        in_specs=[
            pl.BlockSpec((br, d), lambda i: (i, 0)),
            pl.BlockSpec((d, d), lambda i: (0, 0)),
            pl.BlockSpec((br, 1), lambda i: (i, 0)),
        ],
        out_specs=pl.BlockSpec((br, d), lambda i: (i, 0)),
        out_shape=jax.ShapeDtypeStruct((n, d), jnp.float32),
    )(x, w_mat, dinv2d)


def _tc_mid(parts, tp, dinv2d, brow, w_mat):
    n, d = tp.shape
    br = 1000

    def kfn(a_ref, a2_ref, t_ref, dv_ref, bias_ref, w_ref, pre_ref, o_ref):
        pre = ((a_ref[0] + a2_ref[0] + t_ref[...]) * dv_ref[...]
               + bias_ref[...])
        pre_ref[...] = pre
        h = jnp.maximum(pre, 0.0)
        t = jnp.dot(h, w_ref[...], preferred_element_type=jnp.float32)
        o_ref[...] = t * dv_ref[...]

    return pl.pallas_call(
        kfn,
        grid=(n // br,),
        in_specs=[
            pl.BlockSpec((1, br, d), lambda i: (0, i, 0)),
            pl.BlockSpec((1, br, d), lambda i: (1, i, 0)),
            pl.BlockSpec((br, d), lambda i: (i, 0)),
            pl.BlockSpec((br, 1), lambda i: (i, 0)),
            pl.BlockSpec((1, d), lambda i: (0, 0)),
            pl.BlockSpec((d, d), lambda i: (0, 0)),
        ],
        out_specs=[
            pl.BlockSpec((br, d), lambda i: (i, 0)),
            pl.BlockSpec((br, d), lambda i: (i, 0)),
        ],
        out_shape=[
            jax.ShapeDtypeStruct((n, d), jnp.float32),
            jax.ShapeDtypeStruct((n, d), jnp.float32),
        ],
    )(parts, parts, tp, dinv2d, brow, w_mat)


def kernel(x, edge_index, W1, b1, W2, b2, W3, b3):
    n, d = x.shape
    e = edge_index.shape[1]
    ep = pl.cdiv(e, NW * CH) * NW * CH
    ndeg = ((n + CH) // CH) * CH

    pad = ep - e
    hp0 = ((n + 1 + NS * 8 - 1) // (NS * 8)) * NS * 8
    pad_dst = n + jnp.arange(pad, dtype=edge_index.dtype) % (hp0 - n)
    src_pad = jnp.concatenate(
        [edge_index[0], jnp.zeros((pad,), edge_index.dtype)])
    dst_pad = jnp.concatenate([edge_index[1], pad_dst])
    njd = ep // NW // CH
    dstdeg = dst_pad.reshape(NW, njd, CH)

    degp = _deg_fn(ep, ndeg)(dstdeg)
    dinv2d = _tc_dinv(degp, ndeg)[0, :n].reshape(n, 1)
    b1r, b2r, b3r = (b.reshape(1, d) for b in (b1, b2, b3))

    epw = ep // NW
    ea = (int(2 * epw * 0.766) // 128) * 128
    eb = 2 * epw - ea
    hp = ((n + 1 + NS * 8 - 1) // (NS * 8)) * NS * 8
    zro = jnp.zeros((hp, d), jnp.float32)
    scat = _scatter_fn(ep, n, d, ea, eb)

    def run_scatter(t):
        return scat(t, src_pad, dst_pad, zro)

    t1 = _tc_first(x, W1, dinv2d)

    w_stack = jnp.stack([W2, W3, jnp.eye(d, dtype=x.dtype)])
    b_stack = jnp.stack([b1r, b2r, b3r])

    def step(t, xs):
        w_l, b_l = xs
        parts = run_scatter(t)
        pre, t_next = _tc_mid(parts, t, dinv2d, b_l, w_l)
        return t_next, pre

    _, pres = lax.scan(step, t1, (w_stack, b_stack))
    return pres[-1]


# speedup vs baseline: 1.4563x; 1.0228x over previous
import functools

import jax
import jax.numpy as jnp
from jax import lax
from jax.experimental import pallas as pl
from jax.experimental.pallas import tpu as pltpu
from jax.experimental.pallas import tpu_sc as plsc

NC = 2
NS = 16
NW = NC * NS
LANES = 16
CH = 128


def _sc_mesh():
    return plsc.VectorSubcoreMesh(core_axis_name="c", subcore_axis_name="s")


def _sc_params():
    return pltpu.CompilerParams(needs_layout_passes=False)


@functools.lru_cache(maxsize=None)
def _deg_fn(ep, ndeg):
    njd = ep // NW // CH

    @functools.partial(
        pl.kernel,
        out_type=jax.ShapeDtypeStruct((NW, ndeg), jnp.float32),
        mesh=_sc_mesh(),
        compiler_params=_sc_params(),
        scratch_types=[
            pltpu.VMEM((njd, CH), jnp.int32),
            pltpu.VMEM((ndeg,), jnp.float32),
        ],
    )
    def deg_kernel(dst_hbm, out_hbm, dst_v, deg_v):
        c = lax.axis_index("c")
        s = lax.axis_index("s")
        w = s * NC + c
        pltpu.sync_copy(dst_hbm.at[w], dst_v)
        zero16 = jnp.zeros((LANES,), jnp.float32)
        ones16 = jnp.ones((LANES,), jnp.float32)

        def zb(i, carry):
            deg_v[pl.ds(i * LANES, LANES)] = zero16
            return carry

        lax.fori_loop(0, ndeg // LANES, zb, 0)

        def body(j, carry):
            for k in range(CH // LANES):
                idx = dst_v[j, pl.ds(k * LANES, LANES)]
                plsc.addupdate_scatter(deg_v, [idx], ones16)
            return carry

        lax.fori_loop(0, njd, body, 0)
        pltpu.sync_copy(deg_v, out_hbm.at[w])

    return deg_kernel


@functools.lru_cache(maxsize=None)
def _scatter_fn(ep, n, d, ea, eb):
    chs = 64
    hp = ((n + 1 + NS * 8 - 1) // (NS * 8)) * NS * 8
    rt = hp // NS
    emax = max(ea, eb)
    assert 16 * (ea + eb) == ep
    assert ea % (2 * chs) == 0 and eb % (2 * chs) == 0 and rt % 8 == 0

    @functools.partial(
        pl.kernel,
        out_type=jax.ShapeDtypeStruct((NC, hp, d), jnp.float32),
        mesh=_sc_mesh(),
        compiler_params=_sc_params(),
        scratch_types=[
            pltpu.VMEM((emax,), jnp.int32),
            pltpu.VMEM((emax,), jnp.int32),
            pltpu.VMEM((chs, d), jnp.float32),
            pltpu.VMEM((chs, d), jnp.float32),
            pltpu.VMEM((8, d), jnp.float32),
            pltpu.VMEM_SHARED((hp, d), jnp.float32),
            pltpu.SemaphoreType.DMA,
            pltpu.SemaphoreType.DMA,
        ],
    )
    def scat_kernel(t_hbm, src_hbm, dst_hbm, out_hbm,
                    src_v, dst_v, rows_a, rows_b, zbuf, acc, sem_a, sem_b):
        c = lax.axis_index("c")
        s = lax.axis_index("s")

        def gather(j, buf, sem):
            pltpu.async_copy(
                t_hbm.at[src_v.at[pl.ds(j * chs, chs)]], buf, sem)

        def gwait(buf, sem):
            pltpu.make_async_copy(
                t_hbm.at[src_v.at[pl.ds(0, chs)]], buf, sem).wait()

        def scat(j, buf):
            pltpu.sync_copy(buf, acc.at[dst_v.at[pl.ds(j * chs, chs)]],
                            add=True)

        zero16 = jnp.zeros((LANES,), jnp.float32)

        def zb(r, carry):
            for kk in range(d // LANES):
                zbuf[r, pl.ds(kk * LANES, LANES)] = zero16
            return carry

        def run(esz, off):
            nj = esz // chs
            pltpu.sync_copy(src_hbm.at[pl.ds(off, esz)],
                            src_v.at[pl.ds(0, esz)])
            pltpu.sync_copy(dst_hbm.at[pl.ds(off, esz)],
                            dst_v.at[pl.ds(0, esz)])
            lax.fori_loop(0, 8, zb, 0)

            def zc(z, carry):
                pltpu.sync_copy(zbuf, acc.at[pl.ds(s * rt + z * 8, 8)])
                return carry

            lax.fori_loop(0, rt // 8, zc, 0)
            plsc.subcore_barrier()
            gather(0, rows_a, sem_a)

            def body(i, carry):
                j0 = 2 * i
                gather(j0 + 1, rows_b, sem_b)
                gwait(rows_a, sem_a)
                scat(j0, rows_a)

                @pl.when(j0 + 2 < nj)
                def _():
                    gather(j0 + 2, rows_a, sem_a)

                gwait(rows_b, sem_b)
                scat(j0 + 1, rows_b)
                return carry

            lax.fori_loop(0, nj // 2, body, 0)
            plsc.subcore_barrier()
            pltpu.sync_copy(acc.at[pl.ds(s * rt, rt)],
                            out_hbm.at[c, pl.ds(s * rt, rt)])

        @pl.when(c == 0)
        def _():
            run(ea, s * ea)

        @pl.when(c == 1)
        def _():
            run(eb, NS * ea + s * eb)

    return scat_kernel


def _tc_dinv(deg_partial, ndeg):
    def kfn(dp_ref, out_ref):
        sdeg = jnp.sum(dp_ref[...], axis=0, keepdims=True) + 1.0
        out_ref[...] = lax.rsqrt(sdeg)

    return pl.pallas_call(
        kfn,
        out_shape=jax.ShapeDtypeStruct((1, ndeg), jnp.float32),
    )(deg_partial)


def _tc_first(x, w_mat, dinv2d):
    n, d = x.shape
    br = 1000

    def kfn(h_ref, w_ref, dv_ref, o_ref):
        t = jnp.dot(h_ref[...], w_ref[...], preferred_element_type=jnp.float32)
        o_ref[...] = t * dv_ref[...]

    return pl.pallas_call(
        kfn,
        grid=(n // br,),
        in_specs=[
            pl.BlockSpec((br, d), lambda i: (i, 0)),
            pl.BlockSpec((d, d), lambda i: (0, 0)),
            pl.BlockSpec((br, 1), lambda i: (i, 0)),
        ],
        out_specs=pl.BlockSpec((br, d), lambda i: (i, 0)),
        out_shape=jax.ShapeDtypeStruct((n, d), jnp.float32),
    )(x, w_mat, dinv2d)


def _tc_mid(parts, tp, dinv2d, brow, w_mat):
    n, d = tp.shape
    br = 1000

    def kfn(a_ref, a2_ref, t_ref, dv_ref, bias_ref, w_ref, pre_ref, o_ref):
        pre = ((a_ref[0] + a2_ref[0] + t_ref[...]) * dv_ref[...]
               + bias_ref[...])
        pre_ref[...] = pre
        h = jnp.maximum(pre, 0.0)
        t = jnp.dot(h, w_ref[...], preferred_element_type=jnp.float32)
        o_ref[...] = t * dv_ref[...]

    return pl.pallas_call(
        kfn,
        grid=(n // br,),
        in_specs=[
            pl.BlockSpec((1, br, d), lambda i: (0, i, 0)),
            pl.BlockSpec((1, br, d), lambda i: (1, i, 0)),
            pl.BlockSpec((br, d), lambda i: (i, 0)),
            pl.BlockSpec((br, 1), lambda i: (i, 0)),
            pl.BlockSpec((1, d), lambda i: (0, 0)),
            pl.BlockSpec((d, d), lambda i: (0, 0)),
        ],
        out_specs=[
            pl.BlockSpec((br, d), lambda i: (i, 0)),
            pl.BlockSpec((br, d), lambda i: (i, 0)),
        ],
        out_shape=[
            jax.ShapeDtypeStruct((n, d), jnp.float32),
            jax.ShapeDtypeStruct((n, d), jnp.float32),
        ],
    )(parts, parts, tp, dinv2d, brow, w_mat)


def kernel(x, edge_index, W1, b1, W2, b2, W3, b3):
    n, d = x.shape
    e = edge_index.shape[1]
    ep = pl.cdiv(e, NW * CH) * NW * CH
    ndeg = ((n + CH) // CH) * CH

    pad = ep - e
    hp0 = ((n + 1 + NS * 8 - 1) // (NS * 8)) * NS * 8
    pad_dst = n + jnp.arange(pad, dtype=edge_index.dtype) % (hp0 - n)
    src_pad = jnp.concatenate(
        [edge_index[0], jnp.zeros((pad,), edge_index.dtype)])
    dst_pad = jnp.concatenate([edge_index[1], pad_dst])
    njd = ep // NW // CH
    dstdeg = dst_pad.reshape(NW, njd, CH)

    degp = _deg_fn(ep, ndeg)(dstdeg)
    dinv2d = _tc_dinv(degp, ndeg)[0, :n].reshape(n, 1)
    b1r, b2r, b3r = (b.reshape(1, d) for b in (b1, b2, b3))

    epw = ep // NW
    ea = (int(2 * epw * 0.766) // 128) * 128
    eb = 2 * epw - ea
    scat = _scatter_fn(ep, n, d, ea, eb)

    def run_scatter(t):
        return scat(t, src_pad, dst_pad)

    t1 = _tc_first(x, W1, dinv2d)

    w_stack = jnp.stack([W2, W3, jnp.eye(d, dtype=x.dtype)])
    b_stack = jnp.stack([b1r, b2r, b3r])

    def step(t, xs):
        w_l, b_l = xs
        parts = run_scatter(t)
        pre, t_next = _tc_mid(parts, t, dinv2d, b_l, w_l)
        return t_next, pre

    _, pres = lax.scan(step, t1, (w_stack, b_stack))
    return pres[-1]
